# baseline (speedup 1.0000x reference)
import functools

import jax
import jax.numpy as jnp
from jax import lax
from jax.experimental import pallas as pl
from jax.experimental.pallas import tpu as pltpu
from jax.experimental.pallas import tpu_sc as plsc

ROWS = 64
COLS = 32768
LANES = 16
CHUNKS = COLS // LANES
UNROLL = 16
SEGS = 4
SEG_ELEMS = COLS // SEGS
SEG_ITERS = CHUNKS // SEGS // UNROLL
ROWS_PER_W = 2
WORKERS = ROWS // ROWS_PER_W

_mesh = plsc.VectorSubcoreMesh(core_axis_name="c", subcore_axis_name="s")


def _shuffle(x, perm):
    return x.at[perm].get(mode="promise_in_bounds")


def _row_argmin(row_ref, lane, copies):
    minvs = tuple(jnp.full((LANES,), jnp.inf, jnp.float32) for _ in range(UNROLL))
    minis = tuple(jnp.zeros((LANES,), jnp.int32) for _ in range(UNROLL))

    for seg in range(SEGS):
        copies[seg].wait()

        def body(i, carry, _seg=seg):
            mvs, mis = carry
            i_abs = i + _seg * SEG_ITERS
            base = i_abs * (UNROLL * LANES)
            ivec = jnp.full((LANES,), 0, jnp.int32) + i_abs
            nv, ni = [], []
            for u in range(UNROLL):
                v = row_ref[pl.ds(base + u * LANES, LANES)]
                lt = v < mvs[u]
                nv.append(jnp.where(lt, v, mvs[u]))
                ni.append(jnp.where(lt, ivec, mis[u]))
            return tuple(nv), tuple(ni)

        minvs, minis = lax.fori_loop(0, SEG_ITERS, body, (minvs, minis))

    mv = minvs[0]
    mi = minis[0] * (UNROLL * LANES) + lane
    for u in range(1, UNROLL):
        idx_u = minis[u] * (UNROLL * LANES) + (lane + u * LANES)
        better = (minvs[u] < mv) | ((minvs[u] == mv) & (idx_u < mi))
        mv = jnp.where(better, minvs[u], mv)
        mi = jnp.where(better, idx_u, mi)

    for off in (8, 4, 2, 1):
        perm = lane ^ off
        mv2 = _shuffle(mv, perm)
        mi2 = _shuffle(mi, perm)
        better = (mv2 < mv) | ((mv2 == mv) & (mi2 < mi))
        mv = jnp.where(better, mv2, mv)
        mi = jnp.where(better, mi2, mi)
    return mi


@functools.partial(
    pl.kernel,
    out_type=jax.ShapeDtypeStruct((WORKERS, LANES), jnp.float32),
    mesh=_mesh,
    scratch_types=[
        pltpu.VMEM((ROWS_PER_W, COLS), jnp.float32),
        pltpu.VMEM((LANES,), jnp.float32),
    ]
    + [pltpu.SemaphoreType.DMA] * (ROWS_PER_W * SEGS),
)
def _argmin_sc(in_hbm, out_hbm, rows_v, out_v, *sems):
    c = lax.axis_index("c")
    s = lax.axis_index("s")
    wid = s * 2 + c
    r0 = wid * ROWS_PER_W
    lane = lax.iota(jnp.int32, LANES)

    copies = []
    for j in range(ROWS_PER_W):
        row_copies = []
        for seg in range(SEGS):
            cp = pltpu.async_copy(
                in_hbm.at[r0 + j, pl.ds(seg * SEG_ELEMS, SEG_ELEMS)],
                rows_v.at[j, pl.ds(seg * SEG_ELEMS, SEG_ELEMS)],
                sems[j * SEGS + seg],
            )
            row_copies.append(cp)
        copies.append(row_copies)

    b0 = _row_argmin(rows_v.at[0], lane, copies[0])
    b1 = _row_argmin(rows_v.at[1], lane, copies[1])

    outvec = jnp.where(
        lane == 0,
        b0.astype(jnp.float32),
        jnp.where(lane == 1, b1.astype(jnp.float32), jnp.float32(0.0)),
    )
    out_v[...] = outvec
    pltpu.sync_copy(out_v, out_hbm.at[wid])


def kernel(inputs):
    padded = _argmin_sc(inputs)
    return padded[:, :ROWS_PER_W].reshape(ROWS, 1)


# speedup vs baseline: 1.0219x; 1.0219x over previous
import functools

import jax
import jax.numpy as jnp
from jax import lax
from jax.experimental import pallas as pl
from jax.experimental.pallas import tpu as pltpu
from jax.experimental.pallas import tpu_sc as plsc

ROWS = 64
COLS = 32768
LANES = 16
CHUNKS = COLS // LANES
UNROLL = 8
SEGS = 2
SEG_ELEMS = COLS // SEGS
SEG_ITERS = CHUNKS // SEGS // UNROLL
ROWS_PER_W = 2
WORKERS = ROWS // ROWS_PER_W

_mesh = plsc.VectorSubcoreMesh(core_axis_name="c", subcore_axis_name="s")


def _shuffle(x, perm):
    return x.at[perm].get(mode="promise_in_bounds")


def _row_argmin(row_ref, lane, copies):
    minvs = tuple(jnp.full((LANES,), jnp.inf, jnp.float32) for _ in range(UNROLL))
    minis = tuple(jnp.zeros((LANES,), jnp.int32) for _ in range(UNROLL))

    for seg in range(SEGS):
        copies[seg].wait()

        def body(i, carry, _seg=seg):
            mvs, mis = carry
            i_abs = i + _seg * SEG_ITERS
            base = i_abs * (UNROLL * LANES)
            ivec = jnp.full((LANES,), 0, jnp.int32) + i_abs
            nv, ni = [], []
            for u in range(UNROLL):
                v = row_ref[pl.ds(base + u * LANES, LANES)]
                lt = v < mvs[u]
                nv.append(jnp.where(lt, v, mvs[u]))
                ni.append(jnp.where(lt, ivec, mis[u]))
            return tuple(nv), tuple(ni)

        minvs, minis = lax.fori_loop(0, SEG_ITERS, body, (minvs, minis))

    mv = minvs[0]
    mi = minis[0] * (UNROLL * LANES) + lane
    for u in range(1, UNROLL):
        idx_u = minis[u] * (UNROLL * LANES) + (lane + u * LANES)
        better = (minvs[u] < mv) | ((minvs[u] == mv) & (idx_u < mi))
        mv = jnp.where(better, minvs[u], mv)
        mi = jnp.where(better, idx_u, mi)

    for off in (8, 4, 2, 1):
        perm = lane ^ off
        mv2 = _shuffle(mv, perm)
        mi2 = _shuffle(mi, perm)
        better = (mv2 < mv) | ((mv2 == mv) & (mi2 < mi))
        mv = jnp.where(better, mv2, mv)
        mi = jnp.where(better, mi2, mi)
    return mi


@functools.partial(
    pl.kernel,
    out_type=jax.ShapeDtypeStruct((WORKERS, LANES), jnp.float32),
    mesh=_mesh,
    scratch_types=[
        pltpu.VMEM((ROWS_PER_W, COLS), jnp.float32),
        pltpu.VMEM((LANES,), jnp.float32),
    ]
    + [pltpu.SemaphoreType.DMA] * (ROWS_PER_W * SEGS),
)
def _argmin_sc(in_hbm, out_hbm, rows_v, out_v, *sems):
    c = lax.axis_index("c")
    s = lax.axis_index("s")
    wid = s * 2 + c
    r0 = wid * ROWS_PER_W
    lane = lax.iota(jnp.int32, LANES)

    copies = []
    for j in range(ROWS_PER_W):
        row_copies = []
        for seg in range(SEGS):
            cp = pltpu.async_copy(
                in_hbm.at[r0 + j, pl.ds(seg * SEG_ELEMS, SEG_ELEMS)],
                rows_v.at[j, pl.ds(seg * SEG_ELEMS, SEG_ELEMS)],
                sems[j * SEGS + seg],
            )
            row_copies.append(cp)
        copies.append(row_copies)

    b0 = _row_argmin(rows_v.at[0], lane, copies[0])
    b1 = _row_argmin(rows_v.at[1], lane, copies[1])

    outvec = jnp.where(
        lane == 0,
        b0.astype(jnp.float32),
        jnp.where(lane == 1, b1.astype(jnp.float32), jnp.float32(0.0)),
    )
    out_v[...] = outvec
    pltpu.sync_copy(out_v, out_hbm.at[wid])


def kernel(inputs):
    padded = _argmin_sc(inputs)
    return padded[:, :ROWS_PER_W].reshape(ROWS, 1)
